# Initial kernel scaffold; baseline (speedup 1.0000x reference)
import functools

import jax
import jax.numpy as jnp
from jax import lax
from jax.experimental import pallas as pl
from jax.experimental.pallas import tpu as pltpu
from jax.experimental.pallas import tpu_sc as plsc

N = 10000
D = 128
N_CLASSES = 40
N_ACC = 10240
K = 128
NW = 32
E_ROWS = 2528
ROWS_PER_W = E_ROWS // NW
N_TILES = 16
TILE_ROWS = N_ACC // N_TILES
DEG_W = 8


def _zero_fill(ref, rows, width):
    zv = jnp.zeros((16,), jnp.float32)

    def body(i, carry):
        for j in range(width // 16):
            ref[i, pl.ds(j * 16, 16)] = zv
        return carry

    lax.fori_loop(0, rows, body, 0)


def _one_fill(ref, rows, width):
    ov = jnp.ones((16,), jnp.float32)

    def body(i, carry):
        for j in range(width // 16):
            ref[i, pl.ds(j * 16, 16)] = ov
        return carry

    lax.fori_loop(0, rows, body, 0)


def _sc_mesh():
    return plsc.VectorSubcoreMesh(core_axis_name="c", subcore_axis_name="s")


def _sc_degree(dst2d):
    @functools.partial(
        pl.kernel,
        out_type=jax.ShapeDtypeStruct((2, N_ACC, DEG_W), jnp.float32),
        mesh=_sc_mesh(),
        scratch_types=[
            pltpu.VMEM((2, K), jnp.int32),
            pltpu.VMEM((K, DEG_W), jnp.float32),
            pltpu.VMEM_SHARED((N_ACC, DEG_W), jnp.float32),
        ],
    )
    def deg_kernel(dst_hbm, out_hbm, di, buf, acc):
        c = lax.axis_index("c")
        s = lax.axis_index("s")
        wid = s * 2 + c
        base = wid * ROWS_PER_W
        tbase = s * TILE_ROWS

        _zero_fill(buf, K, DEG_W)

        def zacc(t, carry):
            pltpu.sync_copy(buf, acc.at[pl.ds(tbase + t * K, K)])
            return carry

        lax.fori_loop(0, TILE_ROWS // K, zacc, 0)
        _one_fill(buf, K, DEG_W)
        plsc.subcore_barrier()

        def body(t, carry):
            pltpu.sync_copy(dst_hbm.at[base + t], di.at[0])
            pltpu.sync_copy(buf, acc.at[di.at[0]], add=True)
            return carry

        lax.fori_loop(0, ROWS_PER_W, body, 0)
        plsc.subcore_barrier()

        def wout(t, carry):
            r = tbase + t * K
            pltpu.sync_copy(acc.at[pl.ds(r, K)], out_hbm.at[c, pl.ds(r, K)])
            return carry

        lax.fori_loop(0, TILE_ROWS // K, wout, 0)

    return deg_kernel(dst2d)


def _sc_aggregate(y, src2d, dst2d):
    @functools.partial(
        pl.kernel,
        out_type=jax.ShapeDtypeStruct((2, N_ACC, D), jnp.float32),
        mesh=_sc_mesh(),
        scratch_types=[
            pltpu.VMEM((K,), jnp.int32),
            pltpu.VMEM((2, K), jnp.int32),
            pltpu.VMEM((K, D), jnp.float32),
            pltpu.VMEM_SHARED((N_ACC, D), jnp.float32),
            pltpu.SemaphoreType.DMA,
        ],
    )
    def agg_kernel(y_hbm, src_hbm, dst_hbm, out_hbm, si, di, rows, acc, sem):
        c = lax.axis_index("c")
        s = lax.axis_index("s")
        wid = s * 2 + c
        base = wid * ROWS_PER_W
        tbase = s * TILE_ROWS

        _zero_fill(rows, K, D)

        def zacc(t, carry):
            pltpu.sync_copy(rows, acc.at[pl.ds(tbase + t * K, K)])
            return carry

        lax.fori_loop(0, TILE_ROWS // K, zacc, 0)
        plsc.subcore_barrier()

        def body(t, carry):
            j = base + t
            pltpu.sync_copy(src_hbm.at[j], si)
            pltpu.sync_copy(dst_hbm.at[j], di.at[0])
            pltpu.async_copy(y_hbm.at[si], rows, sem).wait()
            pltpu.sync_copy(rows, acc.at[di.at[0]], add=True)
            return carry

        lax.fori_loop(0, ROWS_PER_W, body, 0)
        plsc.subcore_barrier()

        def wout(t, carry):
            r = tbase + t * K
            pltpu.sync_copy(acc.at[pl.ds(r, K)], out_hbm.at[c, pl.ds(r, K)])
            return carry

        lax.fori_loop(0, TILE_ROWS // K, wout, 0)

    return agg_kernel(y, src2d, dst2d)


_R = 1024


def _tc_first_kernel(x_ref, w_ref, degp_ref, y_ref, dinv_ref):
    dp = degp_ref[...]
    dinv = lax.rsqrt(dp[0] + dp[1] + 1.0)
    dinv_ref[...] = dinv
    xw = jnp.dot(x_ref[...], w_ref[...], preferred_element_type=jnp.float32)
    y_ref[...] = dinv[:, 0:1] * xw


def _tc_first(x_pad, w0, degp):
    grid = (N_ACC // _R,)
    return pl.pallas_call(
        _tc_first_kernel,
        grid=grid,
        in_specs=[
            pl.BlockSpec((_R, D), lambda i: (i, 0)),
            pl.BlockSpec((D, D), lambda i: (0, 0)),
            pl.BlockSpec((2, _R, DEG_W), lambda i: (0, i, 0)),
        ],
        out_specs=[
            pl.BlockSpec((_R, D), lambda i: (i, 0)),
            pl.BlockSpec((_R, DEG_W), lambda i: (i, 0)),
        ],
        out_shape=[
            jax.ShapeDtypeStruct((N_ACC, D), jnp.float32),
            jax.ShapeDtypeStruct((N_ACC, DEG_W), jnp.float32),
        ],
    )(x_pad, w0, degp)


def _tc_mid_kernel(p_ref, yp_ref, dinv_ref, b_ref, w_ref, o_ref):
    pp = p_ref[...]
    dv = dinv_ref[...][:, 0:1]
    t = dv * (pp[0] + pp[1] + yp_ref[...]) + b_ref[...]
    h = t * jax.nn.sigmoid(t)
    o_ref[...] = dv * jnp.dot(h, w_ref[...], preferred_element_type=jnp.float32)


def _tc_mid(p, y_prev, dinv, b, w):
    grid = (N_ACC // _R,)
    return pl.pallas_call(
        _tc_mid_kernel,
        grid=grid,
        in_specs=[
            pl.BlockSpec((2, _R, D), lambda i: (0, i, 0)),
            pl.BlockSpec((_R, D), lambda i: (i, 0)),
            pl.BlockSpec((_R, DEG_W), lambda i: (i, 0)),
            pl.BlockSpec((1, D), lambda i: (0, 0)),
            pl.BlockSpec((D, D), lambda i: (0, 0)),
        ],
        out_specs=pl.BlockSpec((_R, D), lambda i: (i, 0)),
        out_shape=jax.ShapeDtypeStruct((N_ACC, D), jnp.float32),
    )(p, y_prev, dinv, b, w)


def _tc_final_kernel(p_ref, yp_ref, dinv_ref, b_ref, o_ref):
    pp = p_ref[...]
    dv = dinv_ref[...][:, 0:1]
    t = dv * (pp[0] + pp[1] + yp_ref[...]) + b_ref[...]
    col = lax.broadcasted_iota(jnp.int32, t.shape, 1)
    valid = col < N_CLASSES
    masked = jnp.where(valid, t, -jnp.inf)
    m = jnp.max(masked, axis=1, keepdims=True)
    ssum = jnp.sum(jnp.where(valid, jnp.exp(t - m), 0.0), axis=1, keepdims=True)
    o_ref[...] = t - (jnp.log(ssum) + m)


def _tc_final(p, y_prev, dinv, b):
    grid = (N_ACC // _R,)
    return pl.pallas_call(
        _tc_final_kernel,
        grid=grid,
        in_specs=[
            pl.BlockSpec((2, _R, D), lambda i: (0, i, 0)),
            pl.BlockSpec((_R, D), lambda i: (i, 0)),
            pl.BlockSpec((_R, DEG_W), lambda i: (i, 0)),
            pl.BlockSpec((1, D), lambda i: (0, 0)),
        ],
        out_specs=pl.BlockSpec((_R, D), lambda i: (i, 0)),
        out_shape=jax.ShapeDtypeStruct((N_ACC, D), jnp.float32),
    )(p, y_prev, dinv, b)


def kernel(x, edge_index, W0, b0, W1, b1, W2, b2):
    e = edge_index.shape[1]
    npad = E_ROWS * K - e
    pad_idx = (jnp.arange(npad, dtype=jnp.int32) % (N_ACC - N)) + N
    src2d = jnp.concatenate([edge_index[0], pad_idx]).reshape(E_ROWS, K)
    dst2d = jnp.concatenate([edge_index[1], pad_idx]).reshape(E_ROWS, K)
    x_pad = jnp.pad(x, ((0, N_ACC - N), (0, 0)))
    w2_pad = jnp.pad(W2, ((0, 0), (0, D - N_CLASSES)))
    b0_2d = b0.reshape(1, D)
    b1_2d = b1.reshape(1, D)
    b2_2d = jnp.pad(b2, (0, D - N_CLASSES)).reshape(1, D)

    degp = _sc_degree(dst2d)
    y0, dinv = _tc_first(x_pad, W0, degp)
    p1 = _sc_aggregate(y0, src2d, dst2d)
    y1 = _tc_mid(p1, y0, dinv, b0_2d, W1)
    p2 = _sc_aggregate(y1, src2d, dst2d)
    y2 = _tc_mid(p2, y1, dinv, b1_2d, w2_pad)
    p3 = _sc_aggregate(y2, src2d, dst2d)
    out = _tc_final(p3, y2, dinv, b2_2d)
    return out[:N, :N_CLASSES]


# speedup vs baseline: 13.1414x; 13.1414x over previous
import functools

import jax
import jax.numpy as jnp
from jax import lax
from jax.experimental import pallas as pl
from jax.experimental.pallas import tpu as pltpu
from jax.experimental.pallas import tpu_sc as plsc

N = 10000
D = 128
N_CLASSES = 40
N_ACC = 10240
K = 128
NW = 32
E_ROWS = 2528
ROWS_PER_W = E_ROWS // NW
N_TILES = 16
TILE_ROWS = N_ACC // N_TILES
DEG_W = 128


def _zero_fill(ref, rows, width):
    zv = jnp.zeros((16,), jnp.float32)

    def body(i, carry):
        for j in range(width // 16):
            ref[i, pl.ds(j * 16, 16)] = zv
        return carry

    lax.fori_loop(0, rows, body, 0)


def _one_fill(ref, rows, width):
    ov = jnp.ones((16,), jnp.float32)

    def body(i, carry):
        for j in range(width // 16):
            ref[i, pl.ds(j * 16, 16)] = ov
        return carry

    lax.fori_loop(0, rows, body, 0)


def _sc_mesh():
    return plsc.VectorSubcoreMesh(
        core_axis_name="c", subcore_axis_name="s", num_cores=2, num_subcores=16
    )


def _sc_degree(dst2d, ones2d):
    @functools.partial(
        pl.kernel,
        out_type=jax.ShapeDtypeStruct((2, N_ACC, DEG_W), jnp.float32),
        mesh=_sc_mesh(),
        scratch_types=[
            pltpu.VMEM((2, K), jnp.int32),
            pltpu.VMEM((K, DEG_W), jnp.float32),
            pltpu.VMEM((K, DEG_W), jnp.float32),
            pltpu.VMEM_SHARED((N_ACC, DEG_W), jnp.float32),
        ],
    )
    def deg_kernel(dst_hbm, ones_hbm, out_hbm, di, zbuf, obuf, acc):
        c = lax.axis_index("c")
        s = lax.axis_index("s")
        wid = s * 2 + c
        base = wid * ROWS_PER_W
        tbase = s * TILE_ROWS

        _zero_fill(zbuf, K, DEG_W)
        pltpu.sync_copy(ones_hbm, obuf)

        def zacc(t, carry):
            pltpu.sync_copy(zbuf, acc.at[pl.ds(tbase + t * K, K)])
            return carry

        lax.fori_loop(0, TILE_ROWS // K, zacc, 0)
        plsc.subcore_barrier()

        def body(t, carry):
            pltpu.sync_copy(dst_hbm.at[base + t], di.at[0])
            pltpu.sync_copy(obuf, acc.at[di.at[0]], add=True)
            return carry

        lax.fori_loop(0, ROWS_PER_W, body, 0)
        plsc.subcore_barrier()

        def wout(t, carry):
            r = tbase + t * K
            pltpu.sync_copy(acc.at[pl.ds(r, K)], out_hbm.at[c, pl.ds(r, K)])
            return carry

        lax.fori_loop(0, TILE_ROWS // K, wout, 0)

    return deg_kernel(dst2d, ones2d)


def _sc_aggregate(y, src2d, dst2d):
    @functools.partial(
        pl.kernel,
        out_type=jax.ShapeDtypeStruct((2, N_ACC, D), jnp.float32),
        mesh=_sc_mesh(),
        scratch_types=[
            pltpu.VMEM((K,), jnp.int32),
            pltpu.VMEM((2, K), jnp.int32),
            pltpu.VMEM((K, D), jnp.float32),
            pltpu.VMEM_SHARED((N_ACC, D), jnp.float32),
            pltpu.SemaphoreType.DMA,
        ],
    )
    def agg_kernel(y_hbm, src_hbm, dst_hbm, out_hbm, si, di, rows, acc, sem):
        c = lax.axis_index("c")
        s = lax.axis_index("s")
        wid = s * 2 + c
        base = wid * ROWS_PER_W
        tbase = s * TILE_ROWS

        _zero_fill(rows, K, D)

        def zacc(t, carry):
            pltpu.sync_copy(rows, acc.at[pl.ds(tbase + t * K, K)])
            return carry

        lax.fori_loop(0, TILE_ROWS // K, zacc, 0)
        plsc.subcore_barrier()

        def body(t, carry):
            j = base + t
            pltpu.sync_copy(src_hbm.at[j], si)
            pltpu.sync_copy(dst_hbm.at[j], di.at[0])
            pltpu.async_copy(y_hbm.at[si], rows, sem).wait()
            pltpu.sync_copy(rows, acc.at[di.at[0]], add=True)
            return carry

        lax.fori_loop(0, ROWS_PER_W, body, 0)
        plsc.subcore_barrier()

        def wout(t, carry):
            r = tbase + t * K
            pltpu.sync_copy(acc.at[pl.ds(r, K)], out_hbm.at[c, pl.ds(r, K)])
            return carry

        lax.fori_loop(0, TILE_ROWS // K, wout, 0)

    return agg_kernel(y, src2d, dst2d)


_R = 1024


def _tc_first_kernel(x_ref, w_ref, degp_ref, y_ref, dinv_ref):
    dp = degp_ref[...]
    dinv = lax.rsqrt(dp[0] + dp[1] + 1.0)
    dinv_ref[...] = dinv
    xw = jnp.dot(x_ref[...], w_ref[...], preferred_element_type=jnp.float32)
    y_ref[...] = dinv[:, 0:1] * xw


def _tc_first(x_pad, w0, degp):
    grid = (N_ACC // _R,)
    return pl.pallas_call(
        _tc_first_kernel,
        grid=grid,
        in_specs=[
            pl.BlockSpec((_R, D), lambda i: (i, 0)),
            pl.BlockSpec((D, D), lambda i: (0, 0)),
            pl.BlockSpec((2, _R, DEG_W), lambda i: (0, i, 0)),
        ],
        out_specs=[
            pl.BlockSpec((_R, D), lambda i: (i, 0)),
            pl.BlockSpec((_R, DEG_W), lambda i: (i, 0)),
        ],
        out_shape=[
            jax.ShapeDtypeStruct((N_ACC, D), jnp.float32),
            jax.ShapeDtypeStruct((N_ACC, DEG_W), jnp.float32),
        ],
    )(x_pad, w0, degp)


def _tc_mid_kernel(p_ref, yp_ref, dinv_ref, b_ref, w_ref, o_ref):
    pp = p_ref[...]
    dv = dinv_ref[...][:, 0:1]
    t = dv * (pp[0] + pp[1] + yp_ref[...]) + b_ref[...]
    h = t * jax.nn.sigmoid(t)
    o_ref[...] = dv * jnp.dot(h, w_ref[...], preferred_element_type=jnp.float32)


def _tc_mid(p, y_prev, dinv, b, w):
    grid = (N_ACC // _R,)
    return pl.pallas_call(
        _tc_mid_kernel,
        grid=grid,
        in_specs=[
            pl.BlockSpec((2, _R, D), lambda i: (0, i, 0)),
            pl.BlockSpec((_R, D), lambda i: (i, 0)),
            pl.BlockSpec((_R, DEG_W), lambda i: (i, 0)),
            pl.BlockSpec((1, D), lambda i: (0, 0)),
            pl.BlockSpec((D, D), lambda i: (0, 0)),
        ],
        out_specs=pl.BlockSpec((_R, D), lambda i: (i, 0)),
        out_shape=jax.ShapeDtypeStruct((N_ACC, D), jnp.float32),
    )(p, y_prev, dinv, b, w)


def _tc_final_kernel(p_ref, yp_ref, dinv_ref, b_ref, o_ref):
    pp = p_ref[...]
    dv = dinv_ref[...][:, 0:1]
    t = dv * (pp[0] + pp[1] + yp_ref[...]) + b_ref[...]
    col = lax.broadcasted_iota(jnp.int32, t.shape, 1)
    valid = col < N_CLASSES
    masked = jnp.where(valid, t, -jnp.inf)
    m = jnp.max(masked, axis=1, keepdims=True)
    ssum = jnp.sum(jnp.where(valid, jnp.exp(t - m), 0.0), axis=1, keepdims=True)
    o_ref[...] = t - (jnp.log(ssum) + m)


def _tc_final(p, y_prev, dinv, b):
    grid = (N_ACC // _R,)
    return pl.pallas_call(
        _tc_final_kernel,
        grid=grid,
        in_specs=[
            pl.BlockSpec((2, _R, D), lambda i: (0, i, 0)),
            pl.BlockSpec((_R, D), lambda i: (i, 0)),
            pl.BlockSpec((_R, DEG_W), lambda i: (i, 0)),
            pl.BlockSpec((1, D), lambda i: (0, 0)),
        ],
        out_specs=pl.BlockSpec((_R, D), lambda i: (i, 0)),
        out_shape=jax.ShapeDtypeStruct((N_ACC, D), jnp.float32),
    )(p, y_prev, dinv, b)


def kernel(x, edge_index, W0, b0, W1, b1, W2, b2):
    e = edge_index.shape[1]
    npad = E_ROWS * K - e
    pad_idx = (jnp.arange(npad, dtype=jnp.int32) % (N_ACC - N)) + N
    src2d = jnp.concatenate([edge_index[0], pad_idx]).reshape(E_ROWS, K)
    dst2d = jnp.concatenate([edge_index[1], pad_idx]).reshape(E_ROWS, K)
    x_pad = jnp.pad(x, ((0, N_ACC - N), (0, 0)))
    w2_pad = jnp.pad(W2, ((0, 0), (0, D - N_CLASSES)))
    b0_2d = b0.reshape(1, D)
    b1_2d = b1.reshape(1, D)
    b2_2d = jnp.pad(b2, (0, D - N_CLASSES)).reshape(1, D)

    ones2d = jnp.ones((K, DEG_W), jnp.float32)
    degp = _sc_degree(dst2d, ones2d)
    y0, dinv = _tc_first(x_pad, W0, degp)
    p1 = _sc_aggregate(y0, src2d, dst2d)
    y1 = _tc_mid(p1, y0, dinv, b0_2d, W1)
    p2 = _sc_aggregate(y1, src2d, dst2d)
    y2 = _tc_mid(p2, y1, dinv, b1_2d, w2_pad)
    p3 = _sc_aggregate(y2, src2d, dst2d)
    out = _tc_final(p3, y2, dinv, b2_2d)
    return out[:N, :N_CLASSES]


# speedup vs baseline: 19.3141x; 1.4697x over previous
import functools

import jax
import jax.numpy as jnp
from jax import lax
from jax.experimental import pallas as pl
from jax.experimental.pallas import tpu as pltpu
from jax.experimental.pallas import tpu_sc as plsc

N = 10000
D = 128
N_CLASSES = 40
N_ACC = 10240
K = 128
NW = 32
E_ROWS = 2560
ROWS_PER_W = E_ROWS // NW
N_TILES = 16
TILE_ROWS = N_ACC // N_TILES
DEG_W = 128


def _zero_fill(ref, rows, width):
    zv = jnp.zeros((16,), jnp.float32)

    def body(i, carry):
        for j in range(width // 16):
            ref[i, pl.ds(j * 16, 16)] = zv
        return carry

    lax.fori_loop(0, rows, body, 0)


def _one_fill(ref, rows, width):
    ov = jnp.ones((16,), jnp.float32)

    def body(i, carry):
        for j in range(width // 16):
            ref[i, pl.ds(j * 16, 16)] = ov
        return carry

    lax.fori_loop(0, rows, body, 0)


def _sc_mesh():
    return plsc.VectorSubcoreMesh(
        core_axis_name="c", subcore_axis_name="s", num_cores=2, num_subcores=16
    )


def _sc_degree(dst2d, ones2d):
    @functools.partial(
        pl.kernel,
        out_type=jax.ShapeDtypeStruct((2, N_ACC, DEG_W), jnp.float32),
        mesh=_sc_mesh(),
        scratch_types=[
            pltpu.VMEM((2, K), jnp.int32),
            pltpu.VMEM((K, DEG_W), jnp.float32),
            pltpu.VMEM((K, DEG_W), jnp.float32),
            pltpu.VMEM_SHARED((N_ACC, DEG_W), jnp.float32),
        ],
    )
    def deg_kernel(dst_hbm, ones_hbm, out_hbm, di, zbuf, obuf, acc):
        c = lax.axis_index("c")
        s = lax.axis_index("s")
        wid = s * 2 + c
        base = wid * ROWS_PER_W
        tbase = s * TILE_ROWS

        _zero_fill(zbuf, K, DEG_W)
        pltpu.sync_copy(ones_hbm, obuf)

        def zacc(t, carry):
            pltpu.sync_copy(zbuf, acc.at[pl.ds(tbase + t * K, K)])
            return carry

        lax.fori_loop(0, TILE_ROWS // K, zacc, 0)
        plsc.subcore_barrier()

        def body(t, carry):
            pltpu.sync_copy(dst_hbm.at[base + t], di.at[0])
            pltpu.sync_copy(obuf, acc.at[di.at[0]], add=True)
            return carry

        lax.fori_loop(0, ROWS_PER_W, body, 0)
        plsc.subcore_barrier()

        def wout(t, carry):
            r = tbase + t * K
            pltpu.sync_copy(acc.at[pl.ds(r, K)], out_hbm.at[c, pl.ds(r, K)])
            return carry

        lax.fori_loop(0, TILE_ROWS // K, wout, 0)

    return deg_kernel(dst2d, ones2d)


def _sc_aggregate(y, src2d, dst2d):
    @functools.partial(
        pl.kernel,
        out_type=jax.ShapeDtypeStruct((2, N_ACC, D), jnp.float32),
        mesh=_sc_mesh(),
        scratch_types=[
            pltpu.VMEM((K,), jnp.int32),
            pltpu.VMEM((K,), jnp.int32),
            pltpu.VMEM((2, K), jnp.int32),
            pltpu.VMEM((K, D), jnp.float32),
            pltpu.VMEM((K, D), jnp.float32),
            pltpu.VMEM_SHARED((N_ACC, D), jnp.float32),
            pltpu.SemaphoreType.DMA,
            pltpu.SemaphoreType.DMA,
        ],
    )
    def agg_kernel(y_hbm, src_hbm, dst_hbm, out_hbm, si0, si1, di,
                   rows0, rows1, acc, sem0, sem1):
        c = lax.axis_index("c")
        s = lax.axis_index("s")
        wid = s * 2 + c
        base = wid * ROWS_PER_W
        tbase = s * TILE_ROWS

        _zero_fill(rows0, K, D)

        def zacc(t, carry):
            pltpu.sync_copy(rows0, acc.at[pl.ds(tbase + t * K, K)])
            return carry

        lax.fori_loop(0, TILE_ROWS // K, zacc, 0)

        pltpu.sync_copy(src_hbm.at[base], si0)
        pltpu.async_copy(y_hbm.at[si0], rows0, sem0)
        plsc.subcore_barrier()

        def body(g, carry):
            t = g * 2
            pltpu.sync_copy(src_hbm.at[base + t + 1], si1)
            pltpu.async_copy(y_hbm.at[si1], rows1, sem1)
            pltpu.sync_copy(dst_hbm.at[base + t], di.at[0])
            pltpu.make_async_copy(y_hbm.at[si0], rows0, sem0).wait()
            pltpu.sync_copy(rows0, acc.at[di.at[0]], add=True)

            @pl.when(t + 2 < ROWS_PER_W)
            def _():
                pltpu.sync_copy(src_hbm.at[base + t + 2], si0)
                pltpu.async_copy(y_hbm.at[si0], rows0, sem0)

            pltpu.sync_copy(dst_hbm.at[base + t + 1], di.at[1])
            pltpu.make_async_copy(y_hbm.at[si1], rows1, sem1).wait()
            pltpu.sync_copy(rows1, acc.at[di.at[1]], add=True)
            return carry

        lax.fori_loop(0, ROWS_PER_W // 2, body, 0)
        plsc.subcore_barrier()

        def wout(t, carry):
            r = tbase + t * K
            pltpu.sync_copy(acc.at[pl.ds(r, K)], out_hbm.at[c, pl.ds(r, K)])
            return carry

        lax.fori_loop(0, TILE_ROWS // K, wout, 0)

    return agg_kernel(y, src2d, dst2d)


_R = 1024


def _tc_first_kernel(x_ref, w_ref, degp_ref, y_ref, dinv_ref):
    dp = degp_ref[...]
    dinv = lax.rsqrt(dp[0] + dp[1] + 1.0)
    dinv_ref[...] = dinv
    xw = jnp.dot(x_ref[...], w_ref[...], preferred_element_type=jnp.float32)
    y_ref[...] = dinv[:, 0:1] * xw


def _tc_first(x_pad, w0, degp):
    grid = (N_ACC // _R,)
    return pl.pallas_call(
        _tc_first_kernel,
        grid=grid,
        in_specs=[
            pl.BlockSpec((_R, D), lambda i: (i, 0)),
            pl.BlockSpec((D, D), lambda i: (0, 0)),
            pl.BlockSpec((2, _R, DEG_W), lambda i: (0, i, 0)),
        ],
        out_specs=[
            pl.BlockSpec((_R, D), lambda i: (i, 0)),
            pl.BlockSpec((_R, DEG_W), lambda i: (i, 0)),
        ],
        out_shape=[
            jax.ShapeDtypeStruct((N_ACC, D), jnp.float32),
            jax.ShapeDtypeStruct((N_ACC, DEG_W), jnp.float32),
        ],
    )(x_pad, w0, degp)


def _tc_mid_kernel(p_ref, yp_ref, dinv_ref, b_ref, w_ref, o_ref):
    pp = p_ref[...]
    dv = dinv_ref[...][:, 0:1]
    t = dv * (pp[0] + pp[1] + yp_ref[...]) + b_ref[...]
    h = t * jax.nn.sigmoid(t)
    o_ref[...] = dv * jnp.dot(h, w_ref[...], preferred_element_type=jnp.float32)


def _tc_mid(p, y_prev, dinv, b, w):
    grid = (N_ACC // _R,)
    return pl.pallas_call(
        _tc_mid_kernel,
        grid=grid,
        in_specs=[
            pl.BlockSpec((2, _R, D), lambda i: (0, i, 0)),
            pl.BlockSpec((_R, D), lambda i: (i, 0)),
            pl.BlockSpec((_R, DEG_W), lambda i: (i, 0)),
            pl.BlockSpec((1, D), lambda i: (0, 0)),
            pl.BlockSpec((D, D), lambda i: (0, 0)),
        ],
        out_specs=pl.BlockSpec((_R, D), lambda i: (i, 0)),
        out_shape=jax.ShapeDtypeStruct((N_ACC, D), jnp.float32),
    )(p, y_prev, dinv, b, w)


def _tc_final_kernel(p_ref, yp_ref, dinv_ref, b_ref, o_ref):
    pp = p_ref[...]
    dv = dinv_ref[...][:, 0:1]
    t = dv * (pp[0] + pp[1] + yp_ref[...]) + b_ref[...]
    col = lax.broadcasted_iota(jnp.int32, t.shape, 1)
    valid = col < N_CLASSES
    masked = jnp.where(valid, t, -jnp.inf)
    m = jnp.max(masked, axis=1, keepdims=True)
    ssum = jnp.sum(jnp.where(valid, jnp.exp(t - m), 0.0), axis=1, keepdims=True)
    o_ref[...] = t - (jnp.log(ssum) + m)


def _tc_final(p, y_prev, dinv, b):
    grid = (N_ACC // _R,)
    return pl.pallas_call(
        _tc_final_kernel,
        grid=grid,
        in_specs=[
            pl.BlockSpec((2, _R, D), lambda i: (0, i, 0)),
            pl.BlockSpec((_R, D), lambda i: (i, 0)),
            pl.BlockSpec((_R, DEG_W), lambda i: (i, 0)),
            pl.BlockSpec((1, D), lambda i: (0, 0)),
        ],
        out_specs=pl.BlockSpec((_R, D), lambda i: (i, 0)),
        out_shape=jax.ShapeDtypeStruct((N_ACC, D), jnp.float32),
    )(p, y_prev, dinv, b)


def kernel(x, edge_index, W0, b0, W1, b1, W2, b2):
    e = edge_index.shape[1]
    npad = E_ROWS * K - e
    pad_idx = (jnp.arange(npad, dtype=jnp.int32) % (N_ACC - N)) + N
    src2d = jnp.concatenate([edge_index[0], pad_idx]).reshape(E_ROWS, K)
    dst2d = jnp.concatenate([edge_index[1], pad_idx]).reshape(E_ROWS, K)
    x_pad = jnp.pad(x, ((0, N_ACC - N), (0, 0)))
    w2_pad = jnp.pad(W2, ((0, 0), (0, D - N_CLASSES)))
    b0_2d = b0.reshape(1, D)
    b1_2d = b1.reshape(1, D)
    b2_2d = jnp.pad(b2, (0, D - N_CLASSES)).reshape(1, D)

    ones2d = jnp.ones((K, DEG_W), jnp.float32)
    degp = _sc_degree(dst2d, ones2d)
    y0, dinv = _tc_first(x_pad, W0, degp)
    p1 = _sc_aggregate(y0, src2d, dst2d)
    y1 = _tc_mid(p1, y0, dinv, b0_2d, W1)
    p2 = _sc_aggregate(y1, src2d, dst2d)
    y2 = _tc_mid(p2, y1, dinv, b1_2d, w2_pad)
    p3 = _sc_aggregate(y2, src2d, dst2d)
    out = _tc_final(p3, y2, dinv, b2_2d)
    return out[:N, :N_CLASSES]


# speedup vs baseline: 21.7977x; 1.1286x over previous
import functools

import jax
import jax.numpy as jnp
from jax import lax
from jax.experimental import pallas as pl
from jax.experimental.pallas import tpu as pltpu
from jax.experimental.pallas import tpu_sc as plsc

N = 10000
D = 128
N_CLASSES = 40
N_ACC = 10240
K = 128
NW = 32
E_ROWS = 2560
ROWS_PER_W = E_ROWS // NW
N_TILES = 16
TILE_ROWS = N_ACC // N_TILES
DEG_W = 128


def _zero_fill(ref, rows, width):
    zv = jnp.zeros((16,), jnp.float32)

    def body(i, carry):
        for j in range(width // 16):
            ref[i, pl.ds(j * 16, 16)] = zv
        return carry

    lax.fori_loop(0, rows, body, 0)


def _one_fill(ref, rows, width):
    ov = jnp.ones((16,), jnp.float32)

    def body(i, carry):
        for j in range(width // 16):
            ref[i, pl.ds(j * 16, 16)] = ov
        return carry

    lax.fori_loop(0, rows, body, 0)


def _sc_mesh():
    return plsc.VectorSubcoreMesh(
        core_axis_name="c", subcore_axis_name="s", num_cores=2, num_subcores=16
    )


def _sc_degree(dst2d, ones2d):
    @functools.partial(
        pl.kernel,
        out_type=jax.ShapeDtypeStruct((2, N_ACC, DEG_W), jnp.float32),
        mesh=_sc_mesh(),
        scratch_types=[
            pltpu.VMEM((2, K), jnp.int32),
            pltpu.VMEM((K, DEG_W), jnp.float32),
            pltpu.VMEM((K, DEG_W), jnp.float32),
            pltpu.VMEM_SHARED((N_ACC, DEG_W), jnp.float32),
            [pltpu.SemaphoreType.DMA] * 2,
        ],
    )
    def deg_kernel(dst_hbm, ones_hbm, out_hbm, di, zbuf, obuf, acc, ssems):
        c = lax.axis_index("c")
        s = lax.axis_index("s")
        wid = s * 2 + c
        base = wid * ROWS_PER_W
        tbase = s * TILE_ROWS

        _zero_fill(zbuf, K, DEG_W)
        pltpu.sync_copy(ones_hbm, obuf)

        def zacc(t, carry):
            pltpu.sync_copy(zbuf, acc.at[pl.ds(tbase + t * K, K)])
            return carry

        lax.fori_loop(0, TILE_ROWS // K, zacc, 0)
        plsc.subcore_barrier()

        def swait(b):
            pltpu.make_async_copy(ones_hbm, obuf, ssems[b]).wait()

        def body(g, carry):
            t = g * 2
            for b in range(2):
                @pl.when(g > 0)
                def _():
                    swait(b)

                pltpu.sync_copy(dst_hbm.at[base + t + b], di.at[b])
                pltpu.async_copy(obuf, acc.at[di.at[b]], ssems[b], add=True)
            return carry

        lax.fori_loop(0, ROWS_PER_W // 2, body, 0)
        for b in range(2):
            swait(b)
        plsc.subcore_barrier()

        def wout(t, carry):
            r = tbase + t * K
            pltpu.sync_copy(acc.at[pl.ds(r, K)], out_hbm.at[c, pl.ds(r, K)])
            return carry

        lax.fori_loop(0, TILE_ROWS // K, wout, 0)

    return deg_kernel(dst2d, ones2d)


def _sc_aggregate(y, src2d, dst2d, d=D):
    @functools.partial(
        pl.kernel,
        out_type=jax.ShapeDtypeStruct((2, N_ACC, d), jnp.float32),
        mesh=_sc_mesh(),
        scratch_types=[
            pltpu.VMEM((2, K), jnp.int32),
            pltpu.VMEM((2, K), jnp.int32),
            pltpu.VMEM((K, d), jnp.float32),
            pltpu.VMEM((K, d), jnp.float32),
            pltpu.VMEM_SHARED((N_ACC, d), jnp.float32),
            [pltpu.SemaphoreType.DMA] * 2,
            [pltpu.SemaphoreType.DMA] * 2,
        ],
    )
    def agg_kernel(y_hbm, src_hbm, dst_hbm, out_hbm, si, di,
                   rows0, rows1, acc, gsems, ssems):
        c = lax.axis_index("c")
        s = lax.axis_index("s")
        wid = s * 2 + c
        base = wid * ROWS_PER_W
        tbase = s * TILE_ROWS
        rows = (rows0, rows1)

        _zero_fill(rows0, K, d)

        def zacc(t, carry):
            pltpu.sync_copy(rows0, acc.at[pl.ds(tbase + t * K, K)])
            return carry

        lax.fori_loop(0, TILE_ROWS // K, zacc, 0)
        plsc.subcore_barrier()

        def lwait(sem, b):
            pltpu.make_async_copy(y_hbm.at[pl.ds(0, K)], rows[b], sem).wait()

        def body(g, carry):
            t = g * 2
            for b in range(2):
                pltpu.sync_copy(src_hbm.at[base + t + b], si.at[b])

                @pl.when(g > 0)
                def _():
                    lwait(ssems[b], b)

                pltpu.async_copy(y_hbm.at[si.at[b]], rows[b], gsems[b])
                pltpu.sync_copy(dst_hbm.at[base + t + b], di.at[b])
            for b in range(2):
                lwait(gsems[b], b)
                pltpu.async_copy(rows[b], acc.at[di.at[b]], ssems[b],
                                 add=True)
            return carry

        lax.fori_loop(0, ROWS_PER_W // 2, body, 0)
        for b in range(2):
            lwait(ssems[b], b)
        plsc.subcore_barrier()

        def wout(t, carry):
            r = tbase + t * K
            pltpu.sync_copy(acc.at[pl.ds(r, K)], out_hbm.at[c, pl.ds(r, K)])
            return carry

        lax.fori_loop(0, TILE_ROWS // K, wout, 0)

    return agg_kernel(y, src2d, dst2d)


_R = 1024


def _tc_first_kernel(x_ref, w_ref, degp_ref, y_ref, dinv_ref):
    dp = degp_ref[...]
    dinv = lax.rsqrt(dp[0] + dp[1] + 1.0)
    dinv_ref[...] = dinv
    xw = jnp.dot(x_ref[...], w_ref[...], preferred_element_type=jnp.float32)
    y_ref[...] = dinv[:, 0:1] * xw


def _tc_first(x_pad, w0, degp):
    grid = (N_ACC // _R,)
    return pl.pallas_call(
        _tc_first_kernel,
        grid=grid,
        in_specs=[
            pl.BlockSpec((_R, D), lambda i: (i, 0)),
            pl.BlockSpec((D, D), lambda i: (0, 0)),
            pl.BlockSpec((2, _R, DEG_W), lambda i: (0, i, 0)),
        ],
        out_specs=[
            pl.BlockSpec((_R, D), lambda i: (i, 0)),
            pl.BlockSpec((_R, DEG_W), lambda i: (i, 0)),
        ],
        out_shape=[
            jax.ShapeDtypeStruct((N_ACC, D), jnp.float32),
            jax.ShapeDtypeStruct((N_ACC, DEG_W), jnp.float32),
        ],
    )(x_pad, w0, degp)


def _tc_mid_kernel(p_ref, yp_ref, dinv_ref, b_ref, w_ref, o_ref):
    pp = p_ref[...]
    dv = dinv_ref[...][:, 0:1]
    t = dv * (pp[0] + pp[1] + yp_ref[...]) + b_ref[...]
    h = t * jax.nn.sigmoid(t)
    o_ref[...] = dv * jnp.dot(h, w_ref[...], preferred_element_type=jnp.float32)


def _tc_mid(p, y_prev, dinv, b, w):
    grid = (N_ACC // _R,)
    return pl.pallas_call(
        _tc_mid_kernel,
        grid=grid,
        in_specs=[
            pl.BlockSpec((2, _R, D), lambda i: (0, i, 0)),
            pl.BlockSpec((_R, D), lambda i: (i, 0)),
            pl.BlockSpec((_R, DEG_W), lambda i: (i, 0)),
            pl.BlockSpec((1, D), lambda i: (0, 0)),
            pl.BlockSpec((D, D), lambda i: (0, 0)),
        ],
        out_specs=pl.BlockSpec((_R, D), lambda i: (i, 0)),
        out_shape=jax.ShapeDtypeStruct((N_ACC, D), jnp.float32),
    )(p, y_prev, dinv, b, w)


def _tc_final_kernel(p_ref, yp_ref, dinv_ref, b_ref, o_ref):
    pp = p_ref[...]
    dv = dinv_ref[...][:, 0:1]
    t = dv * (pp[0] + pp[1] + yp_ref[...]) + b_ref[...]
    col = lax.broadcasted_iota(jnp.int32, t.shape, 1)
    valid = col < N_CLASSES
    masked = jnp.where(valid, t, -jnp.inf)
    m = jnp.max(masked, axis=1, keepdims=True)
    ssum = jnp.sum(jnp.where(valid, jnp.exp(t - m), 0.0), axis=1, keepdims=True)
    o_ref[...] = t - (jnp.log(ssum) + m)


def _tc_final(p, y_prev, dinv, b):
    grid = (N_ACC // _R,)
    return pl.pallas_call(
        _tc_final_kernel,
        grid=grid,
        in_specs=[
            pl.BlockSpec((2, _R, D), lambda i: (0, i, 0)),
            pl.BlockSpec((_R, D), lambda i: (i, 0)),
            pl.BlockSpec((_R, DEG_W), lambda i: (i, 0)),
            pl.BlockSpec((1, D), lambda i: (0, 0)),
        ],
        out_specs=pl.BlockSpec((_R, D), lambda i: (i, 0)),
        out_shape=jax.ShapeDtypeStruct((N_ACC, D), jnp.float32),
    )(p, y_prev, dinv, b)


def kernel(x, edge_index, W0, b0, W1, b1, W2, b2):
    e = edge_index.shape[1]
    npad = E_ROWS * K - e
    pad_idx = (jnp.arange(npad, dtype=jnp.int32) % (N_ACC - N)) + N
    src2d = jnp.concatenate([edge_index[0], pad_idx]).reshape(E_ROWS, K)
    dst2d = jnp.concatenate([edge_index[1], pad_idx]).reshape(E_ROWS, K)
    x_pad = jnp.pad(x, ((0, N_ACC - N), (0, 0)))
    w2_pad = jnp.pad(W2, ((0, 0), (0, D - N_CLASSES)))
    b0_2d = b0.reshape(1, D)
    b1_2d = b1.reshape(1, D)
    b2_2d = jnp.pad(b2, (0, D - N_CLASSES)).reshape(1, D)

    ones2d = jnp.ones((K, DEG_W), jnp.float32)
    degp = _sc_degree(dst2d, ones2d)
    y0, dinv = _tc_first(x_pad, W0, degp)
    p1 = _sc_aggregate(y0, src2d, dst2d)
    y1 = _tc_mid(p1, y0, dinv, b0_2d, W1)
    p2 = _sc_aggregate(y1, src2d, dst2d)
    y2 = _tc_mid(p2, y1, dinv, b1_2d, w2_pad)
    p3 = _sc_aggregate(y2, src2d, dst2d)
    out = _tc_final(p3, y2, dinv, b2_2d)
    return out[:N, :N_CLASSES]
